# baseline (speedup 1.0000x reference)
import functools

import jax
import jax.numpy as jnp
from jax.experimental import pallas as pl

_INTERPRET = False


def _pick_block(total, target):
    b = min(total, target)
    while b > 1:
        if total % b == 0 and (b % 8 == 0 or b == total):
            return b
        b -= 1
    return 1


def _prep_x_kernel(relu_x, x_ref, w_ref, o_ref):
    xv = x_ref[...]
    if relu_x:
        xv = jax.nn.relu(xv)
    o_ref[...] = jnp.dot(xv, w_ref[...], preferred_element_type=jnp.float32)


def _prep_x(x, w, relu_x):
    N, n = x.shape
    K = w.shape[1]
    bn = _pick_block(N, 1000)
    return pl.pallas_call(
        functools.partial(_prep_x_kernel, relu_x),
        grid=(N // bn,),
        in_specs=[
            pl.BlockSpec((bn, n), lambda i: (i, 0)),
            pl.BlockSpec((n, K), lambda i: (0, 0)),
        ],
        out_specs=pl.BlockSpec((bn, K), lambda i: (i, 0)),
        out_shape=jax.ShapeDtypeStruct((N, K), jnp.float32),
        interpret=_INTERPRET,
    )(x, w)


def _prep_u_kernel(relu_u, u_ref, w1_ref, b1_ref, w2_ref, b2_ref, o1_ref, o2_ref):
    uv = u_ref[...]
    if relu_u:
        uv = jax.nn.relu(uv)
    o1_ref[...] = jnp.dot(uv, w1_ref[...], preferred_element_type=jnp.float32) + b1_ref[...]
    o2_ref[...] = jnp.dot(uv, w2_ref[...], preferred_element_type=jnp.float32) + b2_ref[...]


def _prep_u(u, w1, b1, w2, b2, relu_u):
    G, g = u.shape
    H1, H2 = w1.shape[1], w2.shape[1]
    full = lambda a, b: pl.BlockSpec((a, b), lambda: (0, 0))
    return pl.pallas_call(
        functools.partial(_prep_u_kernel, relu_u),
        in_specs=[full(G, g), full(g, H1), full(1, H1), full(g, H2), full(1, H2)],
        out_specs=[full(G, H1), full(G, H2)],
        out_shape=[
            jax.ShapeDtypeStruct((G, H1), jnp.float32),
            jax.ShapeDtypeStruct((G, H2), jnp.float32),
        ],
        interpret=_INTERPRET,
    )(u, w1, b1[None, :], w2, b2[None, :])


def _edge_kernel(relu_ea, G, gsum_ref, ea_ref, bro_ref, uu_ref, w1e_ref, w2_ref,
                 b2_ref, o_ref):
    ea = ea_ref[...]
    if relu_ea:
        ea = jax.nn.relu(ea)
    oh = (jax.lax.broadcasted_iota(jnp.int32, (bro_ref.shape[0], G), 1)
          == bro_ref[...]).astype(jnp.float32)
    h = (gsum_ref[...]
         + jnp.dot(ea, w1e_ref[...], preferred_element_type=jnp.float32)
         + jnp.dot(oh, uu_ref[...], preferred_element_type=jnp.float32))
    h = jax.nn.relu(h)
    o_ref[...] = jnp.dot(h, w2_ref[...], preferred_element_type=jnp.float32) + b2_ref[...]


def _edge_stage(gsum, ea, bro2, uu, w1e, w2, b2, relu_ea):
    E, H = gsum.shape
    e = ea.shape[1]
    eo = w2.shape[1]
    G = uu.shape[0]
    be = _pick_block(E, 2000)
    return pl.pallas_call(
        functools.partial(_edge_kernel, relu_ea, G),
        grid=(E // be,),
        in_specs=[
            pl.BlockSpec((be, H), lambda i: (i, 0)),
            pl.BlockSpec((be, e), lambda i: (i, 0)),
            pl.BlockSpec((be, 1), lambda i: (i, 0)),
            pl.BlockSpec((G, H), lambda i: (0, 0)),
            pl.BlockSpec((e, H), lambda i: (0, 0)),
            pl.BlockSpec((H, eo), lambda i: (0, 0)),
            pl.BlockSpec((1, eo), lambda i: (0, 0)),
        ],
        out_specs=pl.BlockSpec((be, eo), lambda i: (i, 0)),
        out_shape=jax.ShapeDtypeStruct((E, eo), jnp.float32),
        interpret=_INTERPRET,
    )(gsum, ea, bro2, uu, w1e, w2, b2[None, :])


def _node_kernel(relu_x, G, x_ref, s1_ref, s2_ref, invc_ref, m_ref, bat_ref,
                 ub_ref, wn1e_ref, bn1_ref, wn2x_ref, wn2agg_ref, wn2b_ref,
                 bn2b_ref, o_ref):
    agg = ((s1_ref[...]
            + jnp.dot(s2_ref[...], wn1e_ref[...], preferred_element_type=jnp.float32))
           * invc_ref[...]
           + m_ref[...] * bn1_ref[...])
    xv = x_ref[...]
    if relu_x:
        xv = jax.nn.relu(xv)
    oh = (jax.lax.broadcasted_iota(jnp.int32, (bat_ref.shape[0], G), 1)
          == bat_ref[...]).astype(jnp.float32)
    p = (jnp.dot(xv, wn2x_ref[...], preferred_element_type=jnp.float32)
         + jnp.dot(agg, wn2agg_ref[...], preferred_element_type=jnp.float32)
         + jnp.dot(oh, ub_ref[...], preferred_element_type=jnp.float32))
    p = jax.nn.relu(p)
    o_ref[...] = jnp.dot(p, wn2b_ref[...], preferred_element_type=jnp.float32) + bn2b_ref[...]


def _node_stage(x, s1, s2, invc, m, bat2, ub, wn1e, bn1, wn2x, wn2agg, wn2b,
                bn2b, relu_x):
    N, n = x.shape
    H1 = s1.shape[1]
    eo = s2.shape[1]
    H2 = wn2b.shape[0]
    no = wn2b.shape[1]
    G = ub.shape[0]
    bn = _pick_block(N, 1000)
    return pl.pallas_call(
        functools.partial(_node_kernel, relu_x, G),
        grid=(N // bn,),
        in_specs=[
            pl.BlockSpec((bn, n), lambda i: (i, 0)),
            pl.BlockSpec((bn, H1), lambda i: (i, 0)),
            pl.BlockSpec((bn, eo), lambda i: (i, 0)),
            pl.BlockSpec((bn, 1), lambda i: (i, 0)),
            pl.BlockSpec((bn, 1), lambda i: (i, 0)),
            pl.BlockSpec((bn, 1), lambda i: (i, 0)),
            pl.BlockSpec((G, H2), lambda i: (0, 0)),
            pl.BlockSpec((eo, H1), lambda i: (0, 0)),
            pl.BlockSpec((1, H1), lambda i: (0, 0)),
            pl.BlockSpec((n, H2), lambda i: (0, 0)),
            pl.BlockSpec((H1, H2), lambda i: (0, 0)),
            pl.BlockSpec((H2, no), lambda i: (0, 0)),
            pl.BlockSpec((1, no), lambda i: (0, 0)),
        ],
        out_specs=pl.BlockSpec((bn, no), lambda i: (i, 0)),
        out_shape=jax.ShapeDtypeStruct((N, no), jnp.float32),
        interpret=_INTERPRET,
    )(x, s1, s2, invc, m, bat2, ub, wn1e, bn1[None, :], wn2x, wn2agg, wn2b,
      bn2b[None, :])


def _glob_kernel(relu_u, u_ref, nm_ref, wg1u_ref, wg1n_ref, bg1_ref, wg2_ref,
                 bg2_ref, o_ref):
    uv = u_ref[...]
    if relu_u:
        uv = jax.nn.relu(uv)
    h = jax.nn.relu(
        jnp.dot(uv, wg1u_ref[...], preferred_element_type=jnp.float32)
        + jnp.dot(nm_ref[...], wg1n_ref[...], preferred_element_type=jnp.float32)
        + bg1_ref[...])
    o_ref[...] = jnp.dot(h, wg2_ref[...], preferred_element_type=jnp.float32) + bg2_ref[...]


def _glob_stage(u, nm, wg1u, wg1n, bg1, wg2, bg2, relu_u):
    G, g = u.shape
    no = nm.shape[1]
    H = wg2.shape[0]
    go = wg2.shape[1]
    full = lambda a, b: pl.BlockSpec((a, b), lambda: (0, 0))
    return pl.pallas_call(
        functools.partial(_glob_kernel, relu_u),
        in_specs=[full(G, g), full(G, no), full(g, H), full(no, H), full(1, H),
                  full(H, go), full(1, go)],
        out_specs=full(G, go),
        out_shape=jax.ShapeDtypeStruct((G, go), jnp.float32),
        interpret=_INTERPRET,
    )(u, nm, wg1u, wg1n, bg1[None, :], wg2, bg2[None, :])


def kernel(x, edge_index, edge_attr, u, batch, params):
    row = edge_index[0].astype(jnp.int32)
    col = edge_index[1].astype(jnp.int32)
    bat = batch.astype(jnp.int32)
    E = row.shape[0]
    N = x.shape[0]
    G = u.shape[0]

    bro2 = bat[row][:, None]
    bat2 = bat[:, None]
    cnt = jax.ops.segment_sum(jnp.ones((E,), jnp.float32), col, num_segments=N)
    invc = (1.0 / jnp.maximum(cnt, 1.0))[:, None]
    m = (cnt > 0).astype(jnp.float32)[:, None]
    gcnt = jax.ops.segment_sum(jnp.ones((N,), jnp.float32), bat, num_segments=G)
    ginv = (1.0 / jnp.maximum(gcnt, 1.0))[:, None]

    ea = edge_attr
    relu_in = False
    for lp in params:
        (W1, b1), (W2, b2) = lp["edge"]
        ((Wn1, bn1),) = lp["n1"]
        (Wn2a, bn2a), (Wn2b, bn2b) = lp["n2"]
        (Wg1, bg1), (Wg2, bg2) = lp["glob"]
        n = x.shape[1]
        g = u.shape[1]
        e = ea.shape[1]
        H = W1.shape[1]
        Hn1 = Wn1.shape[1]

        W1r, W1c, W1e, W1u = W1[:n], W1[n:2 * n], W1[2 * n:2 * n + e], W1[2 * n + e:]
        Wn1x, Wn1e = Wn1[:n], Wn1[n:]
        Wn2x, Wn2agg, Wn2u = Wn2a[:n], Wn2a[n:n + Hn1], Wn2a[n + Hn1:]
        Wg1u, Wg1n = Wg1[:g], Wg1[g:]

        PM = _prep_x(x, jnp.concatenate([W1r, W1c, Wn1x], axis=1), relu_in)
        uu, ub = _prep_u(u, W1u, b1, Wn2u, bn2a, relu_in)

        gsum = PM[row, :H] + PM[col, H:2 * H]
        new_e = _edge_stage(gsum, ea, bro2, uu, W1e, W2, b2, relu_in)

        s1 = jax.ops.segment_sum(PM[row, 2 * H:], col, num_segments=N)
        s2 = jax.ops.segment_sum(new_e, col, num_segments=N)

        new_x = _node_stage(x, s1, s2, invc, m, bat2, ub, Wn1e, bn1, Wn2x,
                            Wn2agg, Wn2b, bn2b, relu_in)

        nm = jax.ops.segment_sum(new_x, bat, num_segments=G) * ginv
        new_u = _glob_stage(u, nm, Wg1u, Wg1n, bg1, Wg2, bg2, relu_in)

        x, ea, u = new_x, new_e, new_u
        relu_in = True

    return x, ea, u


# speedup vs baseline: 19.5307x; 19.5307x over previous
import functools

import jax
import jax.numpy as jnp
from jax.experimental import pallas as pl

_INTERPRET = False


def _pick_block(total, target):
    b = min(total, target)
    while b > 1:
        if total % b == 0 and (b % 8 == 0 or b == total):
            return b
        b -= 1
    return 1


def _prep_x_kernel(relu_x, H, x_ref, w1_ref, w2_ref, w3_ref, o1_ref, o2_ref):
    xv = x_ref[...]
    if relu_x:
        xv = jax.nn.relu(xv)
    o1_ref[:, :H] = jnp.dot(xv, w1_ref[...], preferred_element_type=jnp.float32)
    o1_ref[:, H:] = jnp.dot(xv, w3_ref[...], preferred_element_type=jnp.float32)
    o2_ref[...] = jnp.dot(xv, w2_ref[...], preferred_element_type=jnp.float32)


def _prep_x(x, w1, w2, w3, relu_x):
    N, n = x.shape
    H = w1.shape[1]
    bn = _pick_block(N, 1000)
    wspec = lambda: pl.BlockSpec((n, H), lambda i: (0, 0))
    return pl.pallas_call(
        functools.partial(_prep_x_kernel, relu_x, H),
        grid=(N // bn,),
        in_specs=[pl.BlockSpec((bn, n), lambda i: (i, 0)), wspec(), wspec(), wspec()],
        out_specs=[pl.BlockSpec((bn, 2 * H), lambda i: (i, 0)),
                   pl.BlockSpec((bn, H), lambda i: (i, 0))],
        out_shape=[jax.ShapeDtypeStruct((N, 2 * H), jnp.float32),
                   jax.ShapeDtypeStruct((N, H), jnp.float32)],
        interpret=_INTERPRET,
    )(x, w1, w2, w3)


def _prep_w_kernel(w2_ref, wn1e_ref, b2_ref, ow_ref, ob_ref):
    ow_ref[...] = jnp.dot(w2_ref[...], wn1e_ref[...], preferred_element_type=jnp.float32)
    ob_ref[...] = jnp.dot(b2_ref[...], wn1e_ref[...], preferred_element_type=jnp.float32)


def _prep_w(w2, wn1e, b2):
    H, eo = w2.shape
    Ho = wn1e.shape[1]
    full = lambda a, b: pl.BlockSpec((a, b), lambda: (0, 0))
    return pl.pallas_call(
        _prep_w_kernel,
        in_specs=[full(H, eo), full(eo, Ho), full(1, eo)],
        out_specs=[full(H, Ho), full(1, Ho)],
        out_shape=[jax.ShapeDtypeStruct((H, Ho), jnp.float32),
                   jax.ShapeDtypeStruct((1, Ho), jnp.float32)],
        interpret=_INTERPRET,
    )(w2, wn1e, b2[None, :])


def _prep_u_kernel(relu_u, u_ref, w1_ref, b1_ref, w2_ref, b2_ref, o1_ref, o2_ref):
    uv = u_ref[...]
    if relu_u:
        uv = jax.nn.relu(uv)
    o1_ref[...] = jnp.dot(uv, w1_ref[...], preferred_element_type=jnp.float32) + b1_ref[...]
    o2_ref[...] = jnp.dot(uv, w2_ref[...], preferred_element_type=jnp.float32) + b2_ref[...]


def _prep_u(u, w1, b1, w2, b2, relu_u):
    G, g = u.shape
    H1, H2 = w1.shape[1], w2.shape[1]
    full = lambda a, b: pl.BlockSpec((a, b), lambda: (0, 0))
    return pl.pallas_call(
        functools.partial(_prep_u_kernel, relu_u),
        in_specs=[full(G, g), full(g, H1), full(1, H1), full(g, H2), full(1, H2)],
        out_specs=[full(G, H1), full(G, H2)],
        out_shape=[
            jax.ShapeDtypeStruct((G, H1), jnp.float32),
            jax.ShapeDtypeStruct((G, H2), jnp.float32),
        ],
        interpret=_INTERPRET,
    )(u, w1, b1[None, :], w2, b2[None, :])


def _edge_kernel(relu_ea, G, H, grn_ref, gc_ref, ea_ref, bro_ref, uu_ref,
                 w1e_ref, w2_ref, b2_ref, w2n_ref, b2n_ref, oe_ref, om_ref):
    ea = ea_ref[...]
    if relu_ea:
        ea = jax.nn.relu(ea)
    oh = (jax.lax.broadcasted_iota(jnp.int32, (bro_ref.shape[0], G), 1)
          == bro_ref[...]).astype(jnp.float32)
    h = (grn_ref[:, :H] + gc_ref[...]
         + jnp.dot(ea, w1e_ref[...], preferred_element_type=jnp.float32)
         + jnp.dot(oh, uu_ref[...], preferred_element_type=jnp.float32))
    h = jax.nn.relu(h)
    oe_ref[...] = jnp.dot(h, w2_ref[...], preferred_element_type=jnp.float32) + b2_ref[...]
    om_ref[...] = (grn_ref[:, H:]
                   + jnp.dot(h, w2n_ref[...], preferred_element_type=jnp.float32)
                   + b2n_ref[...])


def _edge_stage(grn, gc, ea, bro2, uu, w1e, w2, b2, w2n, b2n, relu_ea):
    E, H = gc.shape
    e = ea.shape[1]
    eo = w2.shape[1]
    Ho = w2n.shape[1]
    G = uu.shape[0]
    be = _pick_block(E, 1000)
    return pl.pallas_call(
        functools.partial(_edge_kernel, relu_ea, G, H),
        grid=(E // be,),
        in_specs=[
            pl.BlockSpec((be, 2 * H), lambda i: (i, 0)),
            pl.BlockSpec((be, H), lambda i: (i, 0)),
            pl.BlockSpec((be, e), lambda i: (i, 0)),
            pl.BlockSpec((be, 1), lambda i: (i, 0)),
            pl.BlockSpec((G, H), lambda i: (0, 0)),
            pl.BlockSpec((e, H), lambda i: (0, 0)),
            pl.BlockSpec((H, eo), lambda i: (0, 0)),
            pl.BlockSpec((1, eo), lambda i: (0, 0)),
            pl.BlockSpec((H, Ho), lambda i: (0, 0)),
            pl.BlockSpec((1, Ho), lambda i: (0, 0)),
        ],
        out_specs=[pl.BlockSpec((be, eo), lambda i: (i, 0)),
                   pl.BlockSpec((be, Ho), lambda i: (i, 0))],
        out_shape=[jax.ShapeDtypeStruct((E, eo), jnp.float32),
                   jax.ShapeDtypeStruct((E, Ho), jnp.float32)],
        interpret=_INTERPRET,
    )(grn, gc, ea, bro2, uu, w1e, w2, b2[None, :], w2n, b2n)


def _node_kernel(relu_x, G, x_ref, sagg_ref, invc_ref, m_ref, bat_ref,
                 ub_ref, bn1_ref, wn2x_ref, wn2agg_ref, wn2b_ref,
                 bn2b_ref, o_ref):
    agg = sagg_ref[...] * invc_ref[...] + m_ref[...] * bn1_ref[...]
    xv = x_ref[...]
    if relu_x:
        xv = jax.nn.relu(xv)
    oh = (jax.lax.broadcasted_iota(jnp.int32, (bat_ref.shape[0], G), 1)
          == bat_ref[...]).astype(jnp.float32)
    p = (jnp.dot(xv, wn2x_ref[...], preferred_element_type=jnp.float32)
         + jnp.dot(agg, wn2agg_ref[...], preferred_element_type=jnp.float32)
         + jnp.dot(oh, ub_ref[...], preferred_element_type=jnp.float32))
    p = jax.nn.relu(p)
    o_ref[...] = jnp.dot(p, wn2b_ref[...], preferred_element_type=jnp.float32) + bn2b_ref[...]


def _node_stage(x, sagg, invc, m, bat2, ub, bn1, wn2x, wn2agg, wn2b,
                bn2b, relu_x):
    N, n = x.shape
    H1 = sagg.shape[1]
    H2 = wn2b.shape[0]
    no = wn2b.shape[1]
    G = ub.shape[0]
    bn = _pick_block(N, 1000)
    return pl.pallas_call(
        functools.partial(_node_kernel, relu_x, G),
        grid=(N // bn,),
        in_specs=[
            pl.BlockSpec((bn, n), lambda i: (i, 0)),
            pl.BlockSpec((bn, H1), lambda i: (i, 0)),
            pl.BlockSpec((bn, 1), lambda i: (i, 0)),
            pl.BlockSpec((bn, 1), lambda i: (i, 0)),
            pl.BlockSpec((bn, 1), lambda i: (i, 0)),
            pl.BlockSpec((G, H2), lambda i: (0, 0)),
            pl.BlockSpec((1, H1), lambda i: (0, 0)),
            pl.BlockSpec((n, H2), lambda i: (0, 0)),
            pl.BlockSpec((H1, H2), lambda i: (0, 0)),
            pl.BlockSpec((H2, no), lambda i: (0, 0)),
            pl.BlockSpec((1, no), lambda i: (0, 0)),
        ],
        out_specs=pl.BlockSpec((bn, no), lambda i: (i, 0)),
        out_shape=jax.ShapeDtypeStruct((N, no), jnp.float32),
        interpret=_INTERPRET,
    )(x, sagg, invc, m, bat2, ub, bn1[None, :], wn2x, wn2agg, wn2b,
      bn2b[None, :])


def _glob_kernel(relu_u, u_ref, nm_ref, wg1u_ref, wg1n_ref, bg1_ref, wg2_ref,
                 bg2_ref, o_ref):
    uv = u_ref[...]
    if relu_u:
        uv = jax.nn.relu(uv)
    h = jax.nn.relu(
        jnp.dot(uv, wg1u_ref[...], preferred_element_type=jnp.float32)
        + jnp.dot(nm_ref[...], wg1n_ref[...], preferred_element_type=jnp.float32)
        + bg1_ref[...])
    o_ref[...] = jnp.dot(h, wg2_ref[...], preferred_element_type=jnp.float32) + bg2_ref[...]


def _glob_stage(u, nm, wg1u, wg1n, bg1, wg2, bg2, relu_u):
    G, g = u.shape
    no = nm.shape[1]
    H = wg2.shape[0]
    go = wg2.shape[1]
    full = lambda a, b: pl.BlockSpec((a, b), lambda: (0, 0))
    return pl.pallas_call(
        functools.partial(_glob_kernel, relu_u),
        in_specs=[full(G, g), full(G, no), full(g, H), full(no, H), full(1, H),
                  full(H, go), full(1, go)],
        out_specs=full(G, go),
        out_shape=jax.ShapeDtypeStruct((G, go), jnp.float32),
        interpret=_INTERPRET,
    )(u, nm, wg1u, wg1n, bg1[None, :], wg2, bg2[None, :])


def kernel(x, edge_index, edge_attr, u, batch, params):
    row0 = edge_index[0].astype(jnp.int32)
    col0 = edge_index[1].astype(jnp.int32)
    bat = batch.astype(jnp.int32)
    E = row0.shape[0]
    N = x.shape[0]
    G = u.shape[0]

    perm = jnp.argsort(col0)
    col = col0[perm]
    row = row0[perm]
    inv = jnp.argsort(perm)

    bro2 = bat[row][:, None]
    bat2 = bat[:, None]
    cnt = jax.ops.segment_sum(jnp.ones((E,), jnp.float32), col, num_segments=N,
                              indices_are_sorted=True)
    invc = (1.0 / jnp.maximum(cnt, 1.0))[:, None]
    m = (cnt > 0).astype(jnp.float32)[:, None]
    gcnt = jax.ops.segment_sum(jnp.ones((N,), jnp.float32), bat, num_segments=G,
                               indices_are_sorted=True)
    ginv = (1.0 / jnp.maximum(gcnt, 1.0))[:, None]

    ea = edge_attr[perm]
    relu_in = False
    for lp in params:
        (W1, b1), (W2, b2) = lp["edge"]
        ((Wn1, bn1),) = lp["n1"]
        (Wn2a, bn2a), (Wn2b, bn2b) = lp["n2"]
        (Wg1, bg1), (Wg2, bg2) = lp["glob"]
        n = x.shape[1]
        g = u.shape[1]
        e = ea.shape[1]
        H = W1.shape[1]
        Hn1 = Wn1.shape[1]

        W1r, W1c, W1e, W1u = W1[:n], W1[n:2 * n], W1[2 * n:2 * n + e], W1[2 * n + e:]
        Wn1x, Wn1e = Wn1[:n], Wn1[n:]
        Wn2x, Wn2agg, Wn2u = Wn2a[:n], Wn2a[n:n + Hn1], Wn2a[n + Hn1:]
        Wg1u, Wg1n = Wg1[:g], Wg1[g:]

        xrn, xc = _prep_x(x, W1r, W1c, Wn1x, relu_in)
        uu, ub = _prep_u(u, W1u, b1, Wn2u, bn2a, relu_in)
        w2n, b2n = _prep_w(W2, Wn1e, b2)

        grn = xrn[row]
        gc = xc[col]
        new_e, msg = _edge_stage(grn, gc, ea, bro2, uu, W1e, W2, b2, w2n, b2n,
                                 relu_in)

        sagg = jax.ops.segment_sum(msg, col, num_segments=N,
                                   indices_are_sorted=True)

        new_x = _node_stage(x, sagg, invc, m, bat2, ub, bn1, Wn2x,
                            Wn2agg, Wn2b, bn2b, relu_in)

        nm = jax.ops.segment_sum(new_x, bat, num_segments=G,
                                 indices_are_sorted=True) * ginv
        new_u = _glob_stage(u, nm, Wg1u, Wg1n, bg1, Wg2, bg2, relu_in)

        x, ea, u = new_x, new_e, new_u
        relu_in = True

    return x, ea[inv], u
